# baseline (speedup 1.0000x reference)
import functools

import jax
import jax.numpy as jnp
from jax import lax
from jax.experimental import pallas as pl
from jax.experimental.pallas import tpu as pltpu
from jax.experimental.pallas import tpu_sc as plsc

NUM_SPANS = 16384
P = 65536
SPAN_DIM = 256
HIDDEN = 150
ANT_PER_SPAN = 4

CHUNK = 128
PACKED = 128
NCH = 8


PBLK = 2048


def _pack_body(s_ref, out_ref):
    x = s_ref[...]
    lo = lax.bitcast_convert_type(
        x[:, :PACKED].astype(jnp.bfloat16), jnp.uint16).astype(jnp.uint32)
    hi = lax.bitcast_convert_type(
        x[:, PACKED:].astype(jnp.bfloat16), jnp.uint16).astype(jnp.uint32)
    out_ref[...] = lax.bitcast_convert_type(lo | (hi << 16), jnp.float32)


def _pack(span_reps):
    return pl.pallas_call(
        _pack_body,
        grid=(NUM_SPANS // PBLK,),
        in_specs=[pl.BlockSpec((PBLK, SPAN_DIM), lambda i: (i, 0))],
        out_specs=pl.BlockSpec((PBLK, PACKED), lambda i: (i, 0)),
        out_shape=jax.ShapeDtypeStruct((NUM_SPANS, PACKED), jnp.float32),
    )(span_reps)


def _sc_gather(span_packed, mention_ids, antecedent_ids):
    n = mention_ids.shape[0]
    info = plsc.get_sparse_core_info()
    nc, ns = info.num_cores, info.num_subcores
    nw = nc * ns
    per_w = n // nw
    n_chunks = per_w // CHUNK

    mesh = plsc.VectorSubcoreMesh(core_axis_name="c", subcore_axis_name="s")

    @functools.partial(
        pl.kernel,
        mesh=mesh,
        out_type=[
            jax.ShapeDtypeStruct((n, PACKED), jnp.float32),
            jax.ShapeDtypeStruct((n, PACKED), jnp.float32),
        ],
        scratch_types=[
            pltpu.VMEM((CHUNK,), jnp.int32),
            pltpu.VMEM((CHUNK,), jnp.int32),
            pltpu.VMEM((CHUNK, PACKED), jnp.float32),
            pltpu.VMEM((CHUNK, PACKED), jnp.float32),
            pltpu.SemaphoreType.DMA,
            pltpu.SemaphoreType.DMA,
        ],
    )
    def k(span_hbm, mid_hbm, aid_hbm,
          out_m, out_a,
          idx_m, idx_a, rows_m, rows_a,
          sem0, sem1):
        wid = lax.axis_index("s") * nc + lax.axis_index("c")

        def body(c, _):
            base = wid * per_w + c * CHUNK
            pltpu.sync_copy(mid_hbm.at[pl.ds(base, CHUNK)], idx_m)
            pltpu.sync_copy(aid_hbm.at[pl.ds(base, CHUNK)], idx_a)
            d0 = pltpu.async_copy(span_hbm.at[idx_m], rows_m, sem0)
            d1 = pltpu.async_copy(span_hbm.at[idx_a], rows_a, sem1)
            d0.wait()
            d1.wait()
            pltpu.sync_copy(rows_m, out_m.at[pl.ds(base, CHUNK)])
            pltpu.sync_copy(rows_a, out_a.at[pl.ds(base, CHUNK)])
            return _

        lax.fori_loop(0, n_chunks, body, None)

    return k(span_packed, mention_ids, antecedent_ids)


BLK = 2048


def _score_lookup(ids, smat):
    hi = ids >> 7
    lo = ids & 127
    oh_hi = (hi == lax.broadcasted_iota(jnp.int32, (1, 128), 1)).astype(jnp.float32)
    rows = jnp.dot(oh_hi, smat, preferred_element_type=jnp.float32)
    oh_lo = (lo == lax.broadcasted_iota(jnp.int32, (1, 128), 1)).astype(jnp.float32)
    return jnp.sum(rows * oh_lo, axis=1, keepdims=True)


def _ffnn_body(m_ref, a_ref, mid_ref, aid_ref, did_ref, gid_ref, sid_ref,
               smat_ref, dt_ref, gt_ref, st_ref,
               w1ma_ref, w1p_ref, w1d_ref, w1g_ref, w1s_ref,
               b1_ref, w2_ref, b2_ref, w3_ref, b3_ref, out_ref):
    mid = jnp.reshape(mid_ref[...], (BLK, 1))
    aid = jnp.reshape(aid_ref[...], (BLK, 1))
    did = jnp.reshape(did_ref[...], (BLK, 1))
    gid = jnp.reshape(gid_ref[...], (BLK, 1))
    sid = jnp.reshape(sid_ref[...], (BLK, 1))

    mv = lax.bitcast_convert_type(m_ref[...], jnp.int32)
    av = lax.bitcast_convert_type(a_ref[...], jnp.int32)
    m_e = lax.bitcast_convert_type(mv << 16, jnp.float32)
    m_o = lax.bitcast_convert_type(mv & jnp.int32(-65536), jnp.float32)
    a_e = lax.bitcast_convert_type(av << 16, jnp.float32)
    a_o = lax.bitcast_convert_type(av & jnp.int32(-65536), jnp.float32)
    msg = _score_lookup(mid, smat_ref[...])
    asg = _score_lookup(aid, smat_ref[...])
    x_ma = jnp.concatenate([m_e, m_o, a_e, a_o], axis=1)
    h = jnp.dot(x_ma, w1ma_ref[...], preferred_element_type=jnp.float32)
    prod = jnp.concatenate([m_e * a_e, m_o * a_o], axis=1)
    h = h + jnp.dot(prod, w1p_ref[...], preferred_element_type=jnp.float32)
    projd = jnp.dot(dt_ref[...], w1d_ref[...], preferred_element_type=jnp.float32)
    projg = jnp.dot(gt_ref[...], w1g_ref[...], preferred_element_type=jnp.float32)
    projs = jnp.dot(st_ref[...], w1s_ref[...], preferred_element_type=jnp.float32)
    ohd = (did == lax.broadcasted_iota(jnp.int32, (1, 10), 1)).astype(jnp.float32)
    ohg = (gid == lax.broadcasted_iota(jnp.int32, (1, 8), 1)).astype(jnp.float32)
    ohs = (sid == lax.broadcasted_iota(jnp.int32, (1, 3), 1)).astype(jnp.float32)
    h = h + jnp.dot(ohd, projd, preferred_element_type=jnp.float32)
    h = h + jnp.dot(ohg, projg, preferred_element_type=jnp.float32)
    h = h + jnp.dot(ohs, projs, preferred_element_type=jnp.float32)
    h1 = jnp.maximum(h + b1_ref[...], 0.0)
    h2 = jnp.maximum(jnp.dot(h1, w2_ref[...], preferred_element_type=jnp.float32)
                     + b2_ref[...], 0.0)
    s = jnp.dot(h2, w3_ref[...], preferred_element_type=jnp.float32) + b3_ref[...]
    out_ref[...] = jnp.reshape(s + msg + asg, (1, 1, BLK))


def _ffnn(m_rows, a_rows, mid, aid, did, gid, sid,
          smat_split, dist_table, genre_table, speaker_table,
          w1ma, w1p, w1d, w1g, w1s, b1, W2, b2, W3, b3):
    n = m_rows.shape[0]
    grid = (n // BLK,)
    blk2 = lambda i: (i, 0)
    fixed = lambda i: (0, 0)
    return pl.pallas_call(
        _ffnn_body,
        grid=grid,
        in_specs=[
            pl.BlockSpec((BLK, PACKED), blk2),
            pl.BlockSpec((BLK, PACKED), blk2),
            pl.BlockSpec((1, 1, BLK), lambda i: (i, 0, 0)),
            pl.BlockSpec((1, 1, BLK), lambda i: (i, 0, 0)),
            pl.BlockSpec((1, 1, BLK), lambda i: (i, 0, 0)),
            pl.BlockSpec((1, 1, BLK), lambda i: (i, 0, 0)),
            pl.BlockSpec((1, 1, BLK), lambda i: (i, 0, 0)),
            pl.BlockSpec((128, 128), fixed),
            pl.BlockSpec((10, 20), fixed),
            pl.BlockSpec((8, 20), fixed),
            pl.BlockSpec((3, 20), fixed),
            pl.BlockSpec((512, HIDDEN), fixed),
            pl.BlockSpec((256, HIDDEN), fixed),
            pl.BlockSpec((20, HIDDEN), fixed),
            pl.BlockSpec((20, HIDDEN), fixed),
            pl.BlockSpec((20, HIDDEN), fixed),
            pl.BlockSpec((1, HIDDEN), fixed),
            pl.BlockSpec((HIDDEN, HIDDEN), fixed),
            pl.BlockSpec((1, HIDDEN), fixed),
            pl.BlockSpec((HIDDEN, 1), fixed),
            pl.BlockSpec((1, 1), fixed),
        ],
        out_specs=pl.BlockSpec((1, 1, BLK), lambda i: (i, 0, 0)),
        out_shape=jax.ShapeDtypeStruct((n // BLK, 1, BLK), jnp.float32),
    )(m_rows, a_rows, mid, aid, did, gid, sid,
      smat_split, dist_table, genre_table, speaker_table,
      w1ma, w1p, w1d, w1g, w1s,
      b1.reshape(1, HIDDEN), W2, b2.reshape(1, HIDDEN), W3, b3.reshape(1, 1))


SBLK = 2048


def _softmax_body(s_ref, out_ref):
    x = s_ref[...]
    mx = jnp.maximum(jnp.max(x, axis=1, keepdims=True), 0.0)
    e = jnp.exp(x - mx)
    e0 = jnp.exp(-mx)
    d = jnp.sum(e, axis=1, keepdims=True) + e0
    out_ref[...] = jnp.concatenate([e, e0], axis=1) / d


def _softmax(scores):
    return pl.pallas_call(
        _softmax_body,
        grid=(NUM_SPANS // SBLK,),
        in_specs=[pl.BlockSpec((SBLK, ANT_PER_SPAN), lambda i: (i, 0))],
        out_specs=pl.BlockSpec((SBLK, ANT_PER_SPAN + 1), lambda i: (i, 0)),
        out_shape=jax.ShapeDtypeStruct((NUM_SPANS, ANT_PER_SPAN + 1),
                                       jnp.float32),
    )(scores)


def kernel(span_representations, mention_scores, mention_ids, antecedent_ids,
           distance_ids, genre_ids, speaker_ids,
           dist_table, genre_table, speaker_table,
           W1, b1, W2, b2, W3, b3):
    mid = mention_ids.astype(jnp.int32)
    aid = antecedent_ids.astype(jnp.int32)
    did = distance_ids.astype(jnp.int32)
    gid = genre_ids.astype(jnp.int32)
    sid = speaker_ids.astype(jnp.int32)
    span_packed = _pack(span_representations)
    smat_split = mention_scores.reshape(128, 128)
    w1ma = W1[:2 * SPAN_DIM]
    w1p = W1[2 * SPAN_DIM:3 * SPAN_DIM]
    w1d = W1[768:788]
    w1g = W1[788:808]
    w1s = W1[808:828]

    cp = P // NCH
    s_parts = []
    for c in range(NCH):
        sl = slice(c * cp, (c + 1) * cp)
        m_rows, a_rows = _sc_gather(span_packed, mid[sl], aid[sl])
        s = _ffnn(m_rows, a_rows,
                  mid[sl].reshape(cp // BLK, 1, BLK),
                  aid[sl].reshape(cp // BLK, 1, BLK),
                  did[sl].reshape(cp // BLK, 1, BLK),
                  gid[sl].reshape(cp // BLK, 1, BLK),
                  sid[sl].reshape(cp // BLK, 1, BLK),
                  smat_split, dist_table, genre_table, speaker_table,
                  w1ma, w1p, w1d, w1g, w1s, b1, W2, b2, W3, b3)
        s_parts.append(s)
    s = jnp.concatenate(s_parts, axis=0)
    return _softmax(s.reshape(NUM_SPANS, ANT_PER_SPAN))


# speedup vs baseline: 1.0677x; 1.0677x over previous
import functools

import jax
import jax.numpy as jnp
from jax import lax
from jax.experimental import pallas as pl
from jax.experimental.pallas import tpu as pltpu
from jax.experimental.pallas import tpu_sc as plsc

NUM_SPANS = 16384
P = 65536
SPAN_DIM = 256
HIDDEN = 150
ANT_PER_SPAN = 4

CHUNK = 128
PACKED = 128
NCH = 4


PBLK = 2048


def _pack_body(s_ref, out_ref):
    x = s_ref[...]
    lo = lax.bitcast_convert_type(
        x[:, :PACKED].astype(jnp.bfloat16), jnp.uint16).astype(jnp.uint32)
    hi = lax.bitcast_convert_type(
        x[:, PACKED:].astype(jnp.bfloat16), jnp.uint16).astype(jnp.uint32)
    out_ref[...] = lax.bitcast_convert_type(lo | (hi << 16), jnp.float32)


def _pack(span_reps):
    return pl.pallas_call(
        _pack_body,
        grid=(NUM_SPANS // PBLK,),
        in_specs=[pl.BlockSpec((PBLK, SPAN_DIM), lambda i: (i, 0))],
        out_specs=pl.BlockSpec((PBLK, PACKED), lambda i: (i, 0)),
        out_shape=jax.ShapeDtypeStruct((NUM_SPANS, PACKED), jnp.float32),
    )(span_reps)


def _sc_gather(span_packed, mention_ids, antecedent_ids):
    n = mention_ids.shape[0]
    info = plsc.get_sparse_core_info()
    nc, ns = info.num_cores, info.num_subcores
    nw = nc * ns
    per_w = n // nw
    n_chunks = per_w // CHUNK

    mesh = plsc.VectorSubcoreMesh(core_axis_name="c", subcore_axis_name="s")

    @functools.partial(
        pl.kernel,
        mesh=mesh,
        out_type=[
            jax.ShapeDtypeStruct((n, PACKED), jnp.float32),
            jax.ShapeDtypeStruct((n, PACKED), jnp.float32),
        ],
        scratch_types=[
            pltpu.VMEM((CHUNK,), jnp.int32),
            pltpu.VMEM((CHUNK,), jnp.int32),
            pltpu.VMEM((CHUNK, PACKED), jnp.float32),
            pltpu.VMEM((CHUNK, PACKED), jnp.float32),
            pltpu.SemaphoreType.DMA,
            pltpu.SemaphoreType.DMA,
        ],
    )
    def k(span_hbm, mid_hbm, aid_hbm,
          out_m, out_a,
          idx_m, idx_a, rows_m, rows_a,
          sem0, sem1):
        wid = lax.axis_index("s") * nc + lax.axis_index("c")

        def body(c, _):
            base = wid * per_w + c * CHUNK
            pltpu.sync_copy(mid_hbm.at[pl.ds(base, CHUNK)], idx_m)
            pltpu.sync_copy(aid_hbm.at[pl.ds(base, CHUNK)], idx_a)
            d0 = pltpu.async_copy(span_hbm.at[idx_m], rows_m, sem0)
            d1 = pltpu.async_copy(span_hbm.at[idx_a], rows_a, sem1)
            d0.wait()
            d1.wait()
            pltpu.sync_copy(rows_m, out_m.at[pl.ds(base, CHUNK)])
            pltpu.sync_copy(rows_a, out_a.at[pl.ds(base, CHUNK)])
            return _

        lax.fori_loop(0, n_chunks, body, None)

    return k(span_packed, mention_ids, antecedent_ids)


BLK = 2048


def _score_lookup(ids, smat):
    hi = ids >> 7
    lo = ids & 127
    oh_hi = (hi == lax.broadcasted_iota(jnp.int32, (1, 128), 1)).astype(jnp.float32)
    rows = jnp.dot(oh_hi, smat, preferred_element_type=jnp.float32)
    oh_lo = (lo == lax.broadcasted_iota(jnp.int32, (1, 128), 1)).astype(jnp.float32)
    return jnp.sum(rows * oh_lo, axis=1, keepdims=True)


def _ffnn_body(m_ref, a_ref, mid_ref, aid_ref, did_ref, gid_ref, sid_ref,
               smat_ref, dt_ref, gt_ref, st_ref,
               w1ma_ref, w1p_ref, w1d_ref, w1g_ref, w1s_ref,
               b1_ref, w2_ref, b2_ref, w3_ref, b3_ref, out_ref):
    mid = jnp.reshape(mid_ref[...], (BLK, 1))
    aid = jnp.reshape(aid_ref[...], (BLK, 1))
    did = jnp.reshape(did_ref[...], (BLK, 1))
    gid = jnp.reshape(gid_ref[...], (BLK, 1))
    sid = jnp.reshape(sid_ref[...], (BLK, 1))

    mv = lax.bitcast_convert_type(m_ref[...], jnp.int32)
    av = lax.bitcast_convert_type(a_ref[...], jnp.int32)
    m_e = lax.bitcast_convert_type(mv << 16, jnp.float32)
    m_o = lax.bitcast_convert_type(mv & jnp.int32(-65536), jnp.float32)
    a_e = lax.bitcast_convert_type(av << 16, jnp.float32)
    a_o = lax.bitcast_convert_type(av & jnp.int32(-65536), jnp.float32)
    msg = _score_lookup(mid, smat_ref[...])
    asg = _score_lookup(aid, smat_ref[...])
    x_ma = jnp.concatenate([m_e, m_o, a_e, a_o], axis=1)
    h = jnp.dot(x_ma, w1ma_ref[...], preferred_element_type=jnp.float32)
    prod = jnp.concatenate([m_e * a_e, m_o * a_o], axis=1)
    h = h + jnp.dot(prod, w1p_ref[...], preferred_element_type=jnp.float32)
    projd = jnp.dot(dt_ref[...], w1d_ref[...], preferred_element_type=jnp.float32)
    projg = jnp.dot(gt_ref[...], w1g_ref[...], preferred_element_type=jnp.float32)
    projs = jnp.dot(st_ref[...], w1s_ref[...], preferred_element_type=jnp.float32)
    ohd = (did == lax.broadcasted_iota(jnp.int32, (1, 10), 1)).astype(jnp.float32)
    ohg = (gid == lax.broadcasted_iota(jnp.int32, (1, 8), 1)).astype(jnp.float32)
    ohs = (sid == lax.broadcasted_iota(jnp.int32, (1, 3), 1)).astype(jnp.float32)
    h = h + jnp.dot(ohd, projd, preferred_element_type=jnp.float32)
    h = h + jnp.dot(ohg, projg, preferred_element_type=jnp.float32)
    h = h + jnp.dot(ohs, projs, preferred_element_type=jnp.float32)
    h1 = jnp.maximum(h + b1_ref[...], 0.0)
    h2 = jnp.maximum(jnp.dot(h1, w2_ref[...], preferred_element_type=jnp.float32)
                     + b2_ref[...], 0.0)
    s = jnp.dot(h2, w3_ref[...], preferred_element_type=jnp.float32) + b3_ref[...]
    out_ref[...] = jnp.reshape(s + msg + asg, (1, 1, BLK))


def _ffnn(m_rows, a_rows, mid, aid, did, gid, sid,
          smat_split, dist_table, genre_table, speaker_table,
          w1ma, w1p, w1d, w1g, w1s, b1, W2, b2, W3, b3):
    n = m_rows.shape[0]
    grid = (n // BLK,)
    blk2 = lambda i: (i, 0)
    fixed = lambda i: (0, 0)
    return pl.pallas_call(
        _ffnn_body,
        grid=grid,
        in_specs=[
            pl.BlockSpec((BLK, PACKED), blk2),
            pl.BlockSpec((BLK, PACKED), blk2),
            pl.BlockSpec((1, 1, BLK), lambda i: (i, 0, 0)),
            pl.BlockSpec((1, 1, BLK), lambda i: (i, 0, 0)),
            pl.BlockSpec((1, 1, BLK), lambda i: (i, 0, 0)),
            pl.BlockSpec((1, 1, BLK), lambda i: (i, 0, 0)),
            pl.BlockSpec((1, 1, BLK), lambda i: (i, 0, 0)),
            pl.BlockSpec((128, 128), fixed),
            pl.BlockSpec((10, 20), fixed),
            pl.BlockSpec((8, 20), fixed),
            pl.BlockSpec((3, 20), fixed),
            pl.BlockSpec((512, HIDDEN), fixed),
            pl.BlockSpec((256, HIDDEN), fixed),
            pl.BlockSpec((20, HIDDEN), fixed),
            pl.BlockSpec((20, HIDDEN), fixed),
            pl.BlockSpec((20, HIDDEN), fixed),
            pl.BlockSpec((1, HIDDEN), fixed),
            pl.BlockSpec((HIDDEN, HIDDEN), fixed),
            pl.BlockSpec((1, HIDDEN), fixed),
            pl.BlockSpec((HIDDEN, 1), fixed),
            pl.BlockSpec((1, 1), fixed),
        ],
        out_specs=pl.BlockSpec((1, 1, BLK), lambda i: (i, 0, 0)),
        out_shape=jax.ShapeDtypeStruct((n // BLK, 1, BLK), jnp.float32),
    )(m_rows, a_rows, mid, aid, did, gid, sid,
      smat_split, dist_table, genre_table, speaker_table,
      w1ma, w1p, w1d, w1g, w1s,
      b1.reshape(1, HIDDEN), W2, b2.reshape(1, HIDDEN), W3, b3.reshape(1, 1))


SBLK = 2048


def _softmax_body(s_ref, out_ref):
    x = s_ref[...]
    mx = jnp.maximum(jnp.max(x, axis=1, keepdims=True), 0.0)
    e = jnp.exp(x - mx)
    e0 = jnp.exp(-mx)
    d = jnp.sum(e, axis=1, keepdims=True) + e0
    out_ref[...] = jnp.concatenate([e, e0], axis=1) / d


def _softmax(scores):
    return pl.pallas_call(
        _softmax_body,
        grid=(NUM_SPANS // SBLK,),
        in_specs=[pl.BlockSpec((SBLK, ANT_PER_SPAN), lambda i: (i, 0))],
        out_specs=pl.BlockSpec((SBLK, ANT_PER_SPAN + 1), lambda i: (i, 0)),
        out_shape=jax.ShapeDtypeStruct((NUM_SPANS, ANT_PER_SPAN + 1),
                                       jnp.float32),
    )(scores)


def kernel(span_representations, mention_scores, mention_ids, antecedent_ids,
           distance_ids, genre_ids, speaker_ids,
           dist_table, genre_table, speaker_table,
           W1, b1, W2, b2, W3, b3):
    mid = mention_ids.astype(jnp.int32)
    aid = antecedent_ids.astype(jnp.int32)
    did = distance_ids.astype(jnp.int32)
    gid = genre_ids.astype(jnp.int32)
    sid = speaker_ids.astype(jnp.int32)
    span_packed = _pack(span_representations)
    smat_split = mention_scores.reshape(128, 128)
    w1ma = W1[:2 * SPAN_DIM]
    w1p = W1[2 * SPAN_DIM:3 * SPAN_DIM]
    w1d = W1[768:788]
    w1g = W1[788:808]
    w1s = W1[808:828]

    cp = P // NCH
    s_parts = []
    for c in range(NCH):
        sl = slice(c * cp, (c + 1) * cp)
        m_rows, a_rows = _sc_gather(span_packed, mid[sl], aid[sl])
        s = _ffnn(m_rows, a_rows,
                  mid[sl].reshape(cp // BLK, 1, BLK),
                  aid[sl].reshape(cp // BLK, 1, BLK),
                  did[sl].reshape(cp // BLK, 1, BLK),
                  gid[sl].reshape(cp // BLK, 1, BLK),
                  sid[sl].reshape(cp // BLK, 1, BLK),
                  smat_split, dist_table, genre_table, speaker_table,
                  w1ma, w1p, w1d, w1g, w1s, b1, W2, b2, W3, b3)
        s_parts.append(s)
    s = jnp.concatenate(s_parts, axis=0)
    return _softmax(s.reshape(NUM_SPANS, ANT_PER_SPAN))
